# baseline (speedup 1.0000x reference)
import jax
import jax.numpy as jnp
from jax import lax
from jax.experimental import pallas as pl
from jax.experimental.pallas import tpu as pltpu

_LOG2E = 1.4426950408889634
_LN2 = 0.6931471805599453


def _yolo_body(cls_ref, t_ref, ob_ref, ig_ref, lp_ref, lt_ref,
               out_ref, accv_ref, accl_ref, accl2_ref, acc_ref):
    i = pl.program_id(0)
    g = pl.num_programs(0)
    s = lax.rem(i, 8)
    q = lax.div(i, 8)

    x = cls_ref[0]
    C = x.shape[0]
    tl = t_ref[q, pl.ds(s, 1), :]

    posm = (tl != 0).astype(jnp.float32)

    @pl.when(i == 0)
    def _init():
        accv_ref[...] = jnp.zeros_like(accv_ref)
        accl_ref[...] = jnp.zeros_like(accl_ref)
        accl2_ref[...] = jnp.zeros_like(accl2_ref)
        acc_ref[0] = 0.0
        acc_ref[1] = 0.0

    @pl.when(s == 0)
    def _obj():
        t8 = t_ref[q]
        ob = ob_ref[q]
        ig = ig_ref[q]
        pm = (t8 != 0).astype(jnp.float32)
        spo = jnp.maximum(ob, 0.0) + _LN2 * jnp.log2(
            1.0 + jnp.exp2(jnp.abs(ob) * (-_LOG2E)))
        contrib = pm * (spo - ob) + (1.0 - pm) * (1.0 - ig) * spo
        acc_ref[0] += jnp.sum(contrib)
        acc_ref[1] += jnp.sum(pm)

    bf = jnp.bfloat16
    xb = x.astype(bf)
    y = jnp.exp2(jnp.abs(xb) * bf(-_LOG2E))
    w = bf(1.0) + y * posm.astype(bf)
    z = (w[0:16] * w[16:32]) * (w[32:48] * w[48:64]) * w[64:80]
    z32 = z.astype(jnp.float32)
    accl2_ref[...] += jnp.log2(z32[0:8] * z32[8:16])

    iosub = (lax.broadcasted_iota(jnp.int32, (C, 1), 0) + 1).astype(bf)
    tlb = tl.astype(bf)
    selx = jnp.where(iosub == tlb, xb, bf(0.0))
    m = jnp.maximum(xb, bf(0.0)) - selx

    d = lp_ref[0] - lt_ref[0]
    accl_ref[...] += (d * d) * (0.5 * posm)

    m16 = ((m[0:16] + m[16:32]) + (m[32:48] + m[48:64])
           + m[64:80]).astype(jnp.float32)
    accv_ref[...] += (m16[0:8] + m16[8:16]) * posm

    @pl.when(i == g - 1)
    def _fin():
        total = (jnp.sum(accv_ref[...]) + jnp.sum(accl_ref[...])
                 + _LN2 * jnp.sum(accl2_ref[...]) + acc_ref[0])
        out_ref[...] = jnp.full((1, 1), total / acc_ref[1],
                                dtype=jnp.float32)


def kernel(loc_p, obj_p, cls_p, loc_t, cls_t, ignore):
    B, N, C = cls_p.shape
    assert B % 8 == 0
    NB = B // 8

    t3 = cls_t.reshape(NB, 8, N)
    ob3 = obj_p.reshape(NB, 8, N)
    ig3 = ignore.astype(jnp.float32).reshape(NB, 8, N)

    xT = jnp.transpose(cls_p, (0, 2, 1))
    lpT = jnp.transpose(loc_p, (0, 2, 1))
    ltT = jnp.transpose(loc_t, (0, 2, 1))

    out = pl.pallas_call(
        _yolo_body,
        grid=(B,),
        in_specs=[
            pl.BlockSpec((1, C, N), lambda i: (i, 0, 0)),
            pl.BlockSpec((NB, 8, N), lambda i: (0, 0, 0)),
            pl.BlockSpec((NB, 8, N), lambda i: (0, 0, 0)),
            pl.BlockSpec((NB, 8, N), lambda i: (0, 0, 0)),
            pl.BlockSpec((1, 4, N), lambda i: (i, 0, 0)),
            pl.BlockSpec((1, 4, N), lambda i: (i, 0, 0)),
        ],
        out_specs=pl.BlockSpec((1, 1), lambda i: (0, 0)),
        out_shape=jax.ShapeDtypeStruct((1, 1), jnp.float32),
        scratch_shapes=[pltpu.VMEM((8, N), jnp.float32),
                        pltpu.VMEM((4, N), jnp.float32),
                        pltpu.VMEM((8, N), jnp.float32),
                        pltpu.SMEM((2,), jnp.float32)],
    )(xT, t3, ob3, ig3, lpT, ltT)
    return out[0, 0]


# speedup vs baseline: 1.0348x; 1.0348x over previous
import jax
import jax.numpy as jnp
from jax import lax
from jax.experimental import pallas as pl
from jax.experimental.pallas import tpu as pltpu

_LOG2E = 1.4426950408889634
_LN2 = 0.6931471805599453


def _yolo_body(cls_ref, t_ref, ob_ref, ig_ref, lp_ref, lt_ref,
               out_ref, accv_ref, accl_ref, accl2_ref, acc_ref):
    i = pl.program_id(0)
    g = pl.num_programs(0)
    s = lax.rem(i, 8)

    x = cls_ref[0]
    C = x.shape[0]
    tl = t_ref[0, pl.ds(s, 1), :]

    posm = (tl != 0).astype(jnp.float32)

    @pl.when(i == 0)
    def _init():
        accv_ref[...] = jnp.zeros_like(accv_ref)
        accl_ref[...] = jnp.zeros_like(accl_ref)
        accl2_ref[...] = jnp.zeros_like(accl2_ref)
        acc_ref[0] = 0.0
        acc_ref[1] = 0.0

    @pl.when(s == 0)
    def _obj():
        t8 = t_ref[0]
        ob = ob_ref[0]
        ig = ig_ref[0]
        pm = (t8 != 0).astype(jnp.float32)
        spo = jnp.maximum(ob, 0.0) + _LN2 * jnp.log2(
            1.0 + jnp.exp2(jnp.abs(ob) * (-_LOG2E)))
        contrib = pm * (spo - ob) + (1.0 - pm) * (1.0 - ig) * spo
        acc_ref[0] += jnp.sum(contrib)
        acc_ref[1] += jnp.sum(pm)

    bf = jnp.bfloat16
    xb = x.astype(bf)
    y = jnp.exp2(jnp.abs(xb) * bf(-_LOG2E))
    w = bf(1.0) + y * posm.astype(bf)
    z = (w[0:16] * w[16:32]) * (w[32:48] * w[48:64]) * w[64:80]
    z32 = z.astype(jnp.float32)
    accl2_ref[...] += jnp.log2(z32[0:8] * z32[8:16])

    iosub = (lax.broadcasted_iota(jnp.int32, (C, 1), 0) + 1).astype(bf)
    tlb = tl.astype(bf)
    selx = jnp.where(iosub == tlb, xb, bf(0.0))
    m = jnp.maximum(xb, bf(0.0)) - selx

    d = lp_ref[0] - lt_ref[0]
    accl_ref[...] += (d * d) * (0.5 * posm)

    m16 = ((m[0:16] + m[16:32]) + (m[32:48] + m[48:64])
           + m[64:80]).astype(jnp.float32)
    accv_ref[...] += (m16[0:8] + m16[8:16]) * posm

    @pl.when(i == g - 1)
    def _fin():
        total = (jnp.sum(accv_ref[...]) + jnp.sum(accl_ref[...])
                 + _LN2 * jnp.sum(accl2_ref[...]) + acc_ref[0])
        out_ref[...] = jnp.full((1, 1), total / acc_ref[1],
                                dtype=jnp.float32)


def kernel(loc_p, obj_p, cls_p, loc_t, cls_t, ignore):
    B, N, C = cls_p.shape
    assert B % 8 == 0
    NB = B // 8

    t3 = cls_t.reshape(NB, 8, N)
    ob3 = obj_p.reshape(NB, 8, N)
    ig3 = ignore.astype(jnp.float32).reshape(NB, 8, N)

    xT = jnp.transpose(cls_p, (0, 2, 1))
    lpT = jnp.transpose(loc_p, (0, 2, 1))
    ltT = jnp.transpose(loc_t, (0, 2, 1))

    out = pl.pallas_call(
        _yolo_body,
        grid=(B,),
        in_specs=[
            pl.BlockSpec((1, C, N), lambda i: (i, 0, 0)),
            pl.BlockSpec((1, 8, N), lambda i: (i // 8, 0, 0)),
            pl.BlockSpec((1, 8, N), lambda i: (i // 8, 0, 0)),
            pl.BlockSpec((1, 8, N), lambda i: (i // 8, 0, 0)),
            pl.BlockSpec((1, 4, N), lambda i: (i, 0, 0)),
            pl.BlockSpec((1, 4, N), lambda i: (i, 0, 0)),
        ],
        out_specs=pl.BlockSpec((1, 1), lambda i: (0, 0)),
        out_shape=jax.ShapeDtypeStruct((1, 1), jnp.float32),
        scratch_shapes=[pltpu.VMEM((8, N), jnp.float32),
                        pltpu.VMEM((4, N), jnp.float32),
                        pltpu.VMEM((8, N), jnp.float32),
                        pltpu.SMEM((2,), jnp.float32)],
    )(xT, t3, ob3, ig3, lpT, ltT)
    return out[0, 0]
